# baseline (speedup 1.0000x reference)
import functools

import jax
import jax.numpy as jnp
from jax.experimental import pallas as pl


def _moe_block_kernel(cm_ref, x_ref, rw_ref, rb_ref, a_ref, b_ref, out_ref,
                      *, rank, halves):
    n_exp = rw_ref.shape[0]
    scaling = 1.0 / rank
    hb = x_ref.shape[0] // halves
    xf = x_ref[...]
    logits = jax.lax.dot_general(
        xf, rw_ref[...], (((1,), (1,)), ((), ())),
        preferred_element_type=jnp.float32) + rb_ref[...]
    m = jnp.max(logits, axis=1, keepdims=True)
    p = jnp.exp(logits - m)
    probs = p / jnp.sum(p, axis=1, keepdims=True)
    pmax_all = jnp.max(probs, axis=1, keepdims=True)
    iota = jax.lax.broadcasted_iota(jnp.int32, probs.shape, 1)
    idx_all = jnp.min(jnp.where(probs == pmax_all, iota, n_exp), axis=1,
                      keepdims=True)
    for c in range(halves):
        rows = pl.ds(c * hb, hb)
        x = x_ref[rows, :]
        idx = idx_all[c * hb:(c + 1) * hb, :]
        pmax = pmax_all[c * hb:(c + 1) * hb, :]
        h = jax.lax.dot_general(
            x.astype(jnp.bfloat16), a_ref[...], (((1,), (1,)), ((), ())),
            preferred_element_type=jnp.float32).astype(jnp.bfloat16)
        pscale = (pmax * scaling).astype(jnp.bfloat16)
        hm = jnp.where(cm_ref[...] == idx, h, jnp.bfloat16(0.0)) * pscale
        o = jax.lax.dot_general(
            hm, b_ref[...],
            (((1,), (1,)), ((), ())), preferred_element_type=jnp.float32)
        out_ref[rows, :] = o


def kernel(x, router_w, router_b, lora_A, lora_B):
    b, s, d = x.shape
    n = b * s
    n_exp, rank, _ = lora_A.shape
    out_dim = lora_B.shape[1]
    er = n_exp * rank
    x_flat = x.reshape(n, d)
    a_all = lora_A.reshape(er, d).astype(jnp.bfloat16)
    b_all = jnp.swapaxes(lora_B, 0, 1).reshape(out_dim, er).astype(jnp.bfloat16)
    colmap = (jnp.arange(er, dtype=jnp.int32) // rank).reshape(1, er)
    tb = 1024 if n % 1024 == 0 else n
    halves = 4 if tb % 4 == 0 else 1
    out = pl.pallas_call(
        functools.partial(_moe_block_kernel, rank=rank, halves=halves),
        grid=(n // tb,),
        in_specs=[
            pl.BlockSpec((1, er), lambda i: (0, 0)),
            pl.BlockSpec((tb, d), lambda i: (i, 0)),
            pl.BlockSpec((n_exp, d), lambda i: (0, 0)),
            pl.BlockSpec((1, n_exp), lambda i: (0, 0)),
            pl.BlockSpec((er, d), lambda i: (0, 0)),
            pl.BlockSpec((out_dim, er), lambda i: (0, 0)),
        ],
        out_specs=pl.BlockSpec((tb, out_dim), lambda i: (i, 0)),
        out_shape=jax.ShapeDtypeStruct((n, out_dim), x.dtype),
    )(colmap, x_flat, router_w, router_b.reshape(1, n_exp), a_all, b_all)
    return out.reshape(b, s, out_dim)


# speedup vs baseline: 1.0814x; 1.0814x over previous
import functools

import jax
import jax.numpy as jnp
from jax.experimental import pallas as pl


def _moe_block_kernel(cm_ref, x_ref, rw_ref, rb_ref, a_ref, b_ref, out_ref,
                      *, rank, halves):
    n_exp = rw_ref.shape[0]
    scaling = 1.0 / rank
    hb = x_ref.shape[0] // halves
    for c in range(halves):
        rows = pl.ds(c * hb, hb)
        x = x_ref[rows, :]
        logits = jax.lax.dot_general(
            x, rw_ref[...], (((1,), (1,)), ((), ())),
            preferred_element_type=jnp.float32) + rb_ref[...]
        m = jnp.max(logits, axis=1, keepdims=True)
        p = jnp.exp(logits - m)
        probs = p / jnp.sum(p, axis=1, keepdims=True)
        pmax = jnp.max(probs, axis=1, keepdims=True)
        iota = jax.lax.broadcasted_iota(jnp.int32, probs.shape, 1)
        idx = jnp.min(jnp.where(probs == pmax, iota, n_exp), axis=1,
                      keepdims=True)
        h = jax.lax.dot_general(
            x.astype(jnp.bfloat16), a_ref[...], (((1,), (1,)), ((), ())),
            preferred_element_type=jnp.float32).astype(jnp.bfloat16)
        pscale = (pmax * scaling).astype(jnp.bfloat16)
        hm = jnp.where(cm_ref[...] == idx, h, jnp.bfloat16(0.0)) * pscale
        o = jax.lax.dot_general(
            hm, b_ref[...],
            (((1,), (1,)), ((), ())), preferred_element_type=jnp.float32)
        out_ref[rows, :] = o


def kernel(x, router_w, router_b, lora_A, lora_B):
    b, s, d = x.shape
    n = b * s
    n_exp, rank, _ = lora_A.shape
    out_dim = lora_B.shape[1]
    er = n_exp * rank
    x_flat = x.reshape(n, d)
    a_all = lora_A.reshape(er, d).astype(jnp.bfloat16)
    b_all = jnp.swapaxes(lora_B, 0, 1).reshape(out_dim, er).astype(jnp.bfloat16)
    colmap = (jnp.arange(er, dtype=jnp.int32) // rank).reshape(1, er)
    tb = 1024 if n % 1024 == 0 else n
    halves = 4 if tb % 4 == 0 else 1
    out = pl.pallas_call(
        functools.partial(_moe_block_kernel, rank=rank, halves=halves),
        grid=(n // tb,),
        in_specs=[
            pl.BlockSpec((1, er), lambda i: (0, 0)),
            pl.BlockSpec((tb, d), lambda i: (i, 0)),
            pl.BlockSpec((n_exp, d), lambda i: (0, 0)),
            pl.BlockSpec((1, n_exp), lambda i: (0, 0)),
            pl.BlockSpec((er, d), lambda i: (0, 0)),
            pl.BlockSpec((out_dim, er), lambda i: (0, 0)),
        ],
        out_specs=pl.BlockSpec((tb, out_dim), lambda i: (i, 0)),
        out_shape=jax.ShapeDtypeStruct((n, out_dim), x.dtype),
    )(colmap, x_flat, router_w, router_b.reshape(1, n_exp), a_all, b_all)
    return out.reshape(b, s, out_dim)


# speedup vs baseline: 1.0833x; 1.0017x over previous
import functools

import jax
import jax.numpy as jnp
from jax.experimental import pallas as pl
from jax.experimental.pallas import tpu as pltpu


def _moe_block_kernel(cm_ref, x_ref, rw_ref, rb_ref, a_ref, b_ref, out_ref,
                      *, rank, chunks):
    n_exp = rw_ref.shape[0]
    scaling = 1.0 / rank
    hb = x_ref.shape[0] // chunks
    for c in range(chunks):
        rows = pl.ds(c * hb, hb)
        x = x_ref[rows, :]
        logits = jax.lax.dot_general(
            x, rw_ref[...], (((1,), (1,)), ((), ())),
            preferred_element_type=jnp.float32) + rb_ref[...]
        m = jnp.max(logits, axis=1, keepdims=True)
        p = jnp.exp(logits - m)
        probs = p / jnp.sum(p, axis=1, keepdims=True)
        pmax = jnp.max(probs, axis=1, keepdims=True)
        iota = jax.lax.broadcasted_iota(jnp.int32, probs.shape, 1)
        idx = jnp.min(jnp.where(probs == pmax, iota, n_exp), axis=1,
                      keepdims=True)
        h = jax.lax.dot_general(
            x.astype(jnp.bfloat16), a_ref[...], (((1,), (1,)), ((), ())),
            preferred_element_type=jnp.float32).astype(jnp.bfloat16)
        pscale = (pmax * scaling).astype(jnp.bfloat16)
        hm = jnp.where(cm_ref[...] == idx, h, jnp.bfloat16(0.0)) * pscale
        o = jax.lax.dot_general(
            hm, b_ref[...],
            (((1,), (1,)), ((), ())), preferred_element_type=jnp.float32)
        out_ref[rows, :] = o


def kernel(x, router_w, router_b, lora_A, lora_B):
    b, s, d = x.shape
    n = b * s
    n_exp, rank, _ = lora_A.shape
    out_dim = lora_B.shape[1]
    er = n_exp * rank
    x_flat = x.reshape(n, d)
    a_all = lora_A.reshape(er, d).astype(jnp.bfloat16)
    b_all = jnp.swapaxes(lora_B, 0, 1).reshape(out_dim, er).astype(jnp.bfloat16)
    colmap = (jnp.arange(er, dtype=jnp.int32) // rank).reshape(1, er)
    tb = 1024 if n % 1024 == 0 else n
    chunks = 4 if tb % 4 == 0 else 1
    out = pl.pallas_call(
        functools.partial(_moe_block_kernel, rank=rank, chunks=chunks),
        grid=(n // tb,),
        in_specs=[
            pl.BlockSpec((1, er), lambda i: (0, 0)),
            pl.BlockSpec((tb, d), lambda i: (i, 0)),
            pl.BlockSpec((n_exp, d), lambda i: (0, 0)),
            pl.BlockSpec((1, n_exp), lambda i: (0, 0)),
            pl.BlockSpec((er, d), lambda i: (0, 0)),
            pl.BlockSpec((out_dim, er), lambda i: (0, 0)),
        ],
        out_specs=pl.BlockSpec((tb, out_dim), lambda i: (i, 0)),
        compiler_params=pltpu.CompilerParams(dimension_semantics=("parallel",)),
        out_shape=jax.ShapeDtypeStruct((n, out_dim), x.dtype),
    )(colmap, x_flat, router_w, router_b.reshape(1, n_exp), a_all, b_all)
    return out.reshape(b, s, out_dim)
